# baseline (speedup 1.0000x reference)
import functools
import jax
import jax.numpy as jnp
from jax import lax
from jax.experimental import pallas as pl
from jax.experimental.pallas import tpu as pltpu
from jax.experimental.pallas import tpu_sc as plsc

N, P, D = 10000, 8, 256
L = 16
T = D // L
NC, NS = 2, 16
NW = NC * NS
CH = 16
NCHUNK = N // CH
BASE_CH = NCHUNK // NW
EXTRA = NCHUNK - BASE_CH * NW
HALF = (BASE_CH - 1) // 2
QUAD = (BASE_CH - 3) // 4


def _worker_start(wid):
    return wid * BASE_CH + jnp.minimum(wid, EXTRA)


def _sc_mesh():
    return plsc.VectorSubcoreMesh(core_axis_name="c", subcore_axis_name="s")


def _in_start(h_hbm, cid, buf, sem):
    pltpu.async_copy(h_hbm.at[pl.ds(cid * CH, CH)], buf, sem)


def _in_wait(h_hbm, buf, sem):
    pltpu.make_async_copy(h_hbm.at[pl.ds(0, CH)], buf, sem).wait()


def _lane_gather(v, idx):
    dnums = lax.GatherDimensionNumbers(
        offset_dims=(), collapsed_slice_dims=(0,), start_index_map=(0,))
    return lax.gather(v, idx[:, None], dnums, (1,),
                      mode=lax.GatherScatterMode.PROMISE_IN_BOUNDS)


def _pass1_body(h_hbm, w_hbm, out_hbm, wbuf, buf0, buf1, stage, sem0, sem1):
    wid = lax.axis_index("s") * NC + lax.axis_index("c")
    pltpu.sync_copy(w_hbm, wbuf)
    wv = [wbuf[pl.ds(t * L, L)] for t in range(T)]
    start = _worker_start(wid)

    def rows(buf, acc):
        def row_body(r, acc):
            acc = list(acc)
            for p in range(P):
                a = acc[p]
                for t in range(T):
                    a = a + buf[r, p, pl.ds(t * L, L)] * wv[t]
                acc[p] = a
            return tuple(acc)
        return lax.fori_loop(0, CH, row_body, acc)

    _in_start(h_hbm, start, buf0, sem0)

    def body2(k, acc):
        g = 2 * k
        _in_start(h_hbm, start + g + 1, buf1, sem1)
        _in_wait(h_hbm, buf0, sem0)
        acc = rows(buf0, acc)
        _in_start(h_hbm, start + g + 2, buf0, sem0)
        _in_wait(h_hbm, buf1, sem1)
        return rows(buf1, acc)

    acc0 = tuple(jnp.zeros((L,), jnp.float32) for _ in range(P))
    acc = lax.fori_loop(0, HALF, body2, acc0)
    _in_wait(h_hbm, buf0, sem0)
    acc = rows(buf0, acc)
    for p in range(P):
        stage[p, :] = acc[p]

    @pl.when(wid < EXTRA)
    def _extra():
        pltpu.sync_copy(h_hbm.at[pl.ds((start + BASE_CH) * CH, CH)], buf1)
        acc_e = rows(buf1, acc0)
        for p in range(P):
            stage[p, :] = stage[p, :] + acc_e[p]

    pltpu.sync_copy(stage, out_hbm.at[wid])


def _pass1(h, Wf):
    f = pl.kernel(
        _pass1_body,
        out_type=jax.ShapeDtypeStruct((NW, P, L), jnp.float32),
        mesh=_sc_mesh(),
        scratch_types=[
            pltpu.VMEM((D,), jnp.float32),
            pltpu.VMEM((CH, P, D), jnp.float32),
            pltpu.VMEM((CH, P, D), jnp.float32),
            pltpu.VMEM((P, L), jnp.float32),
            pltpu.SemaphoreType.DMA,
            pltpu.SemaphoreType.DMA,
        ],
    )
    return f(h, Wf)


def _pass2_body(h_hbm, wpart_hbm, z_hbm, wpbuf, buf0, buf1,
                zq0, zq1, zq2, zq3, sem0, sem1,
                osem0, osem1, osem2, osem3):
    wid = lax.axis_index("s") * NC + lax.axis_index("c")
    start = _worker_start(wid)

    pltpu.sync_copy(wpart_hbm, wpbuf)
    lanes = lax.iota(jnp.int32, L)
    ws = []
    for p in range(P):
        v = wpbuf[0, p, :]
        for w in range(1, NW):
            v = v + wpbuf[w, p, :]
        for sh in (8, 4, 2, 1):
            v = v + _lane_gather(v, lanes ^ sh)
        v = jnp.where(v >= 0.0, v, 0.01 * v)
        ws.append(v)
    wmax = ws[0]
    for p in range(1, P):
        wmax = jnp.maximum(wmax, ws[p])
    ev = [jnp.exp(ws[p] - wmax) for p in range(P)]
    tot = ev[0]
    for p in range(1, P):
        tot = tot + ev[p]
    bv = [ev[p] / tot for p in range(P)]

    zqs = [zq0, zq1, zq2, zq3]
    osems = [osem0, osem1, osem2, osem3]

    def rows(buf, zq):
        def row_body(r, c):
            for t in range(T):
                z = bv[0] * buf[r, 0, pl.ds(t * L, L)]
                for p in range(1, P):
                    z = z + bv[p] * buf[r, p, pl.ds(t * L, L)]
                zq[r, pl.ds(t * L, L)] = z
            return c
        lax.fori_loop(0, CH, row_body, 0)

    def out_start(cid, j):
        pltpu.async_copy(zqs[j], z_hbm.at[pl.ds(cid * CH, CH), :], osems[j])

    def out_wait(j):
        pltpu.make_async_copy(zqs[j], z_hbm.at[pl.ds(0, CH), :], osems[j]).wait()

    def slot(cid, buf, j, first):
        @pl.when(jnp.logical_not(first))
        def _():
            out_wait(j)
        rows(buf, zqs[j])
        out_start(cid, j)

    _in_start(h_hbm, start, buf0, sem0)

    def body4(k, c):
        g = 4 * k
        first = k == 0
        _in_start(h_hbm, start + g + 1, buf1, sem1)
        _in_wait(h_hbm, buf0, sem0)
        slot(start + g, buf0, 0, first)
        _in_start(h_hbm, start + g + 2, buf0, sem0)
        _in_wait(h_hbm, buf1, sem1)
        slot(start + g + 1, buf1, 1, first)
        _in_start(h_hbm, start + g + 3, buf1, sem1)
        _in_wait(h_hbm, buf0, sem0)
        slot(start + g + 2, buf0, 2, first)
        _in_start(h_hbm, start + g + 4, buf0, sem0)
        _in_wait(h_hbm, buf1, sem1)
        slot(start + g + 3, buf1, 3, first)
        return c

    lax.fori_loop(0, QUAD, body4, 0)
    false = jnp.bool_(False)
    _in_start(h_hbm, start + 17, buf1, sem1)
    _in_wait(h_hbm, buf0, sem0)
    slot(start + 16, buf0, 0, false)
    _in_start(h_hbm, start + 18, buf0, sem0)
    _in_wait(h_hbm, buf1, sem1)
    slot(start + 17, buf1, 1, false)
    _in_wait(h_hbm, buf0, sem0)
    slot(start + 18, buf0, 2, false)

    @pl.when(wid < EXTRA)
    def _extra():
        pltpu.sync_copy(h_hbm.at[pl.ds((start + BASE_CH) * CH, CH)], buf1)
        slot(start + BASE_CH, buf1, 3, false)

    for j in range(4):
        out_wait(j)


def _pass2(h, wpart):
    f = pl.kernel(
        _pass2_body,
        out_type=jax.ShapeDtypeStruct((N, D), jnp.float32),
        mesh=_sc_mesh(),
        scratch_types=[
            pltpu.VMEM((NW, P, L), jnp.float32),
            pltpu.VMEM((CH, P, D), jnp.float32),
            pltpu.VMEM((CH, P, D), jnp.float32),
            pltpu.VMEM((CH, D), jnp.float32),
            pltpu.VMEM((CH, D), jnp.float32),
            pltpu.VMEM((CH, D), jnp.float32),
            pltpu.VMEM((CH, D), jnp.float32),
            pltpu.SemaphoreType.DMA,
            pltpu.SemaphoreType.DMA,
            pltpu.SemaphoreType.DMA,
            pltpu.SemaphoreType.DMA,
            pltpu.SemaphoreType.DMA,
            pltpu.SemaphoreType.DMA,
        ],
    )
    return f(h, wpart)


def kernel(h, W):
    Wf = W.reshape(D)
    wpart = _pass1(h, Wf)
    return _pass2(h, wpart)


# speedup vs baseline: 1.4664x; 1.4664x over previous
import functools
import jax
import jax.numpy as jnp
from jax import lax
from jax.experimental import pallas as pl
from jax.experimental.pallas import tpu as pltpu
from jax.experimental.pallas import tpu_sc as plsc

N, P, D = 10000, 8, 256
L = 16
T = D // L
NC, NS = 2, 16
NW = NC * NS
CH = 16

SC_BASE = 7
SC_EXTRA = 16
SC_NCH = NW * SC_BASE + SC_EXTRA
S = SC_NCH * CH
A = N - S
HALF = (SC_BASE - 1) // 2
CH0 = A // CH

BN = 616
NB = A // BN


def _worker_start(wid):
    return CH0 + wid * SC_BASE + jnp.minimum(wid, SC_EXTRA)


def _sc_mesh():
    return plsc.VectorSubcoreMesh(core_axis_name="c", subcore_axis_name="s")


def _in_start(h_hbm, cid, buf, sem):
    pltpu.async_copy(h_hbm.at[pl.ds(cid * CH, CH)], buf, sem)


def _in_wait(h_hbm, buf, sem):
    pltpu.make_async_copy(h_hbm.at[pl.ds(0, CH)], buf, sem).wait()


def _p1sc_body(h_hbm, w_hbm, out_hbm, wbuf, buf0, buf1, stage, sem0, sem1):
    wid = lax.axis_index("s") * NC + lax.axis_index("c")
    pltpu.sync_copy(w_hbm, wbuf)
    wv = [wbuf[pl.ds(t * L, L)] for t in range(T)]
    start = _worker_start(wid)

    def rows(buf, acc):
        def row_body(r, acc):
            acc = list(acc)
            for p in range(P):
                a = acc[p]
                for t in range(T):
                    a = a + buf[r, p, pl.ds(t * L, L)] * wv[t]
                acc[p] = a
            return tuple(acc)
        return lax.fori_loop(0, CH, row_body, acc)

    _in_start(h_hbm, start, buf0, sem0)

    def body2(k, acc):
        g = 2 * k
        _in_start(h_hbm, start + g + 1, buf1, sem1)
        _in_wait(h_hbm, buf0, sem0)
        acc = rows(buf0, acc)
        _in_start(h_hbm, start + g + 2, buf0, sem0)
        _in_wait(h_hbm, buf1, sem1)
        return rows(buf1, acc)

    acc0 = tuple(jnp.zeros((L,), jnp.float32) for _ in range(P))
    acc = lax.fori_loop(0, HALF, body2, acc0)
    _in_wait(h_hbm, buf0, sem0)
    acc = rows(buf0, acc)
    for p in range(P):
        stage[p, :] = acc[p]

    @pl.when(wid < SC_EXTRA)
    def _extra():
        pltpu.sync_copy(h_hbm.at[pl.ds((start + SC_BASE) * CH, CH)], buf1)
        acc_e = rows(buf1, acc0)
        for p in range(P):
            stage[p, :] = stage[p, :] + acc_e[p]

    pltpu.sync_copy(stage, out_hbm.at[wid])


def _p1sc(h, Wf):
    f = pl.kernel(
        _p1sc_body,
        out_type=jax.ShapeDtypeStruct((NW, P, L), jnp.float32),
        mesh=_sc_mesh(),
        scratch_types=[
            pltpu.VMEM((D,), jnp.float32),
            pltpu.VMEM((CH, P, D), jnp.float32),
            pltpu.VMEM((CH, P, D), jnp.float32),
            pltpu.VMEM((P, L), jnp.float32),
            pltpu.SemaphoreType.DMA,
            pltpu.SemaphoreType.DMA,
        ],
    )
    return f(h, Wf)


def _p1tc_body(h_ref, zphi_ref, acc_ref):
    i = pl.program_id(0)

    @pl.when(i == 0)
    def _init():
        acc_ref[...] = jnp.zeros_like(acc_ref)

    acc_ref[...] += jnp.sum(h_ref[...], axis=0)

    @pl.when(i == NB - 1)
    def _fin():
        zphi_ref[...] = acc_ref[...]


def _p1tc(h):
    return pl.pallas_call(
        _p1tc_body,
        grid=(NB,),
        in_specs=[pl.BlockSpec((BN, P, D), lambda i: (i, 0, 0))],
        out_specs=pl.BlockSpec((P, D), lambda i: (0, 0)),
        out_shape=jax.ShapeDtypeStruct((P, D), jnp.float32),
        scratch_shapes=[pltpu.VMEM((P, D), jnp.float32)],
    )(h)


def _beta_body(zphi_ref, w_ref, wpart_ref, beta_sc_ref, beta_tc_ref):
    w = jnp.dot(zphi_ref[...], w_ref[...])
    w = w + jnp.sum(wpart_ref[...], axis=(0, 2)).reshape(P, 1)
    w = jnp.where(w >= 0, w, 0.01 * w)
    m = jnp.max(w, axis=0, keepdims=True)
    e = jnp.exp(w - m)
    beta = e / jnp.sum(e, axis=0, keepdims=True)
    beta_sc_ref[...] = jnp.broadcast_to(beta, (P, 128))
    beta_tc_ref[...] = jnp.broadcast_to(beta, (P, D))


def _beta(zphi, Wm, wpart):
    return pl.pallas_call(
        _beta_body,
        out_shape=(
            jax.ShapeDtypeStruct((P, 128), jnp.float32),
            jax.ShapeDtypeStruct((P, D), jnp.float32),
        ),
    )(zphi, Wm, wpart)


def _p2tc_body(h_ref, beta_ref, z_ref):
    z_ref[...] = jnp.sum(h_ref[...] * beta_ref[...][None, :, :], axis=1)


def _p2tc(h, beta_tc):
    return pl.pallas_call(
        _p2tc_body,
        grid=(NB,),
        in_specs=[
            pl.BlockSpec((BN, P, D), lambda i: (i, 0, 0)),
            pl.BlockSpec((P, D), lambda i: (0, 0)),
        ],
        out_specs=pl.BlockSpec((BN, D), lambda i: (i, 0)),
        out_shape=jax.ShapeDtypeStruct((A, D), jnp.float32),
    )(h, beta_tc)


def _p2sc_body(h_hbm, beta_hbm, z_hbm, bbuf, buf0, buf1, zb0, zb1,
               sem0, sem1, osem0, osem1):
    wid = lax.axis_index("s") * NC + lax.axis_index("c")
    pltpu.sync_copy(beta_hbm, bbuf)
    bv = [bbuf[p, pl.ds(0, L)] for p in range(P)]
    start = _worker_start(wid)

    def rows(buf, zb):
        def row_body(r, c):
            for t in range(T):
                z = bv[0] * buf[r, 0, pl.ds(t * L, L)]
                for p in range(1, P):
                    z = z + bv[p] * buf[r, p, pl.ds(t * L, L)]
                zb[r, pl.ds(t * L, L)] = z
            return c
        lax.fori_loop(0, CH, row_body, 0)

    def out_start(cid, zb, osem):
        pltpu.async_copy(zb, z_hbm.at[pl.ds(cid * CH - A, CH), :], osem)

    def out_wait(zb, osem):
        pltpu.make_async_copy(zb, z_hbm.at[pl.ds(0, CH), :], osem).wait()

    _in_start(h_hbm, start, buf0, sem0)

    def body2(k, c):
        g = 2 * k
        _in_start(h_hbm, start + g + 1, buf1, sem1)
        _in_wait(h_hbm, buf0, sem0)

        @pl.when(k > 0)
        def _():
            out_wait(zb0, osem0)
        rows(buf0, zb0)
        out_start(start + g, zb0, osem0)
        _in_start(h_hbm, start + g + 2, buf0, sem0)
        _in_wait(h_hbm, buf1, sem1)

        @pl.when(k > 0)
        def _():
            out_wait(zb1, osem1)
        rows(buf1, zb1)
        out_start(start + g + 1, zb1, osem1)
        return c

    lax.fori_loop(0, HALF, body2, 0)
    _in_wait(h_hbm, buf0, sem0)
    out_wait(zb0, osem0)
    rows(buf0, zb0)
    out_start(start + SC_BASE - 1, zb0, osem0)

    @pl.when(wid < SC_EXTRA)
    def _extra():
        pltpu.sync_copy(h_hbm.at[pl.ds((start + SC_BASE) * CH, CH)], buf1)
        out_wait(zb1, osem1)
        rows(buf1, zb1)
        out_start(start + SC_BASE, zb1, osem1)

    out_wait(zb0, osem0)
    out_wait(zb1, osem1)


def _p2sc(h, beta_sc):
    f = pl.kernel(
        _p2sc_body,
        out_type=jax.ShapeDtypeStruct((S, D), jnp.float32),
        mesh=_sc_mesh(),
        scratch_types=[
            pltpu.VMEM((P, 128), jnp.float32),
            pltpu.VMEM((CH, P, D), jnp.float32),
            pltpu.VMEM((CH, P, D), jnp.float32),
            pltpu.VMEM((CH, D), jnp.float32),
            pltpu.VMEM((CH, D), jnp.float32),
            pltpu.SemaphoreType.DMA,
            pltpu.SemaphoreType.DMA,
            pltpu.SemaphoreType.DMA,
            pltpu.SemaphoreType.DMA,
        ],
    )
    return f(h, beta_sc)


def kernel(h, W):
    wpart = _p1sc(h, W.reshape(D))
    zphi = _p1tc(h)
    beta_sc, beta_tc = _beta(zphi, W, wpart)
    z_sc = _p2sc(h, beta_sc)
    z_tc = _p2tc(h, beta_tc)
    return jnp.concatenate([z_tc, z_sc], axis=0)


# speedup vs baseline: 1.8050x; 1.2309x over previous
import functools
import jax
import jax.numpy as jnp
from jax import lax
from jax.experimental import pallas as pl
from jax.experimental.pallas import tpu as pltpu
from jax.experimental.pallas import tpu_sc as plsc

N, P, D = 10000, 8, 256
L = 16
T = D // L
NC, NS = 2, 16
NW = NC * NS
CH = 16

SC_BASE = 5
SC_EXTRA = 30
SC_NCH = NW * SC_BASE + SC_EXTRA
S = SC_NCH * CH
A = N - S
HALF = (SC_BASE - 1) // 2
CH0 = A // CH

BN = 696
NB = A // BN
BN2 = 400
NB2 = N // BN2


def _worker_start(wid):
    return CH0 + wid * SC_BASE + jnp.minimum(wid, SC_EXTRA)


def _sc_mesh():
    return plsc.VectorSubcoreMesh(core_axis_name="c", subcore_axis_name="s")


def _in_start(h_hbm, cid, buf, sem):
    pltpu.async_copy(h_hbm.at[pl.ds(cid * CH, CH)], buf, sem)


def _in_wait(h_hbm, buf, sem):
    pltpu.make_async_copy(h_hbm.at[pl.ds(0, CH)], buf, sem).wait()


def _p1sc_body(h_hbm, w_hbm, out_hbm, wbuf, buf0, buf1, stage, sem0, sem1):
    wid = lax.axis_index("s") * NC + lax.axis_index("c")
    pltpu.sync_copy(w_hbm, wbuf)
    wv = [wbuf[pl.ds(t * L, L)] for t in range(T)]
    start = _worker_start(wid)

    def rows(buf, acc):
        def row_body(r, acc):
            acc = list(acc)
            for p in range(P):
                a = acc[p]
                for t in range(T):
                    a = a + buf[r, p, pl.ds(t * L, L)] * wv[t]
                acc[p] = a
            return tuple(acc)
        return lax.fori_loop(0, CH, row_body, acc)

    _in_start(h_hbm, start, buf0, sem0)

    def body2(k, acc):
        g = 2 * k
        _in_start(h_hbm, start + g + 1, buf1, sem1)
        _in_wait(h_hbm, buf0, sem0)
        acc = rows(buf0, acc)
        _in_start(h_hbm, start + g + 2, buf0, sem0)
        _in_wait(h_hbm, buf1, sem1)
        return rows(buf1, acc)

    acc0 = tuple(jnp.zeros((L,), jnp.float32) for _ in range(P))
    acc = lax.fori_loop(0, HALF, body2, acc0)
    _in_wait(h_hbm, buf0, sem0)
    acc = rows(buf0, acc)
    for p in range(P):
        stage[p, :] = acc[p]

    @pl.when(wid < SC_EXTRA)
    def _extra():
        pltpu.sync_copy(h_hbm.at[pl.ds((start + SC_BASE) * CH, CH)], buf1)
        acc_e = rows(buf1, acc0)
        for p in range(P):
            stage[p, :] = stage[p, :] + acc_e[p]

    pltpu.sync_copy(stage, out_hbm.at[wid])


def _p1sc(h, Wf):
    f = pl.kernel(
        _p1sc_body,
        out_type=jax.ShapeDtypeStruct((NW, P, L), jnp.float32),
        mesh=_sc_mesh(),
        scratch_types=[
            pltpu.VMEM((D,), jnp.float32),
            pltpu.VMEM((CH, P, D), jnp.float32),
            pltpu.VMEM((CH, P, D), jnp.float32),
            pltpu.VMEM((P, L), jnp.float32),
            pltpu.SemaphoreType.DMA,
            pltpu.SemaphoreType.DMA,
        ],
    )
    return f(h, Wf)


def _p1tc_body(h_ref, zphi_ref, acc_ref):
    i = pl.program_id(0)

    @pl.when(i == 0)
    def _init():
        acc_ref[...] = jnp.zeros_like(acc_ref)

    acc_ref[...] += jnp.sum(h_ref[...], axis=0)

    @pl.when(i == NB - 1)
    def _fin():
        zphi_ref[...] = acc_ref[...]


def _p1tc(h):
    return pl.pallas_call(
        _p1tc_body,
        grid=(NB,),
        in_specs=[pl.BlockSpec((BN, P, D), lambda i: (i, 0, 0))],
        out_specs=pl.BlockSpec((P, D), lambda i: (0, 0)),
        out_shape=jax.ShapeDtypeStruct((P, D), jnp.float32),
        scratch_shapes=[pltpu.VMEM((P, D), jnp.float32)],
    )(h)


def _beta_body(zphi_ref, w_ref, wpart_ref, beta_tc_ref):
    w = jnp.dot(zphi_ref[...], w_ref[...])
    w = w + jnp.sum(wpart_ref[...], axis=(0, 2)).reshape(P, 1)
    w = jnp.where(w >= 0, w, 0.01 * w)
    m = jnp.max(w, axis=0, keepdims=True)
    e = jnp.exp(w - m)
    beta = e / jnp.sum(e, axis=0, keepdims=True)
    beta_tc_ref[...] = jnp.broadcast_to(beta, (P, D))


def _beta(zphi, Wm, wpart):
    return pl.pallas_call(
        _beta_body,
        out_shape=jax.ShapeDtypeStruct((P, D), jnp.float32),
    )(zphi, Wm, wpart)


def _p2tc_body(h_ref, beta_ref, z_ref):
    z_ref[...] = jnp.sum(h_ref[...] * beta_ref[...][None, :, :], axis=1)


def _p2tc(h, beta_tc):
    return pl.pallas_call(
        _p2tc_body,
        grid=(NB2,),
        in_specs=[
            pl.BlockSpec((BN2, P, D), lambda i: (i, 0, 0)),
            pl.BlockSpec((P, D), lambda i: (0, 0)),
        ],
        out_specs=pl.BlockSpec((BN2, D), lambda i: (i, 0)),
        out_shape=jax.ShapeDtypeStruct((N, D), jnp.float32),
    )(h, beta_tc)


def kernel(h, W):
    wpart = _p1sc(h, W.reshape(D))
    zphi = _p1tc(h)
    beta_tc = _beta(zphi, W, wpart)
    return _p2tc(h, beta_tc)


# speedup vs baseline: 1.9975x; 1.1066x over previous
import functools
import jax
import jax.numpy as jnp
from jax import lax
from jax.experimental import pallas as pl
from jax.experimental.pallas import tpu as pltpu
from jax.experimental.pallas import tpu_sc as plsc

N, P, D = 10000, 8, 256
L = 16
T = D // L
NC, NS = 2, 16
NW = NC * NS
CH = 16

SC_BASE = 5
SC_EXTRA = 30
SC_NCH = NW * SC_BASE + SC_EXTRA
S = SC_NCH * CH
A = N - S
HALF = (SC_BASE - 1) // 2
CH0 = A // CH

BN = 464
NB = A // BN
BN2 = 1000
NB2 = N // BN2


def _worker_start(wid):
    return CH0 + wid * SC_BASE + jnp.minimum(wid, SC_EXTRA)


def _sc_mesh():
    return plsc.VectorSubcoreMesh(core_axis_name="c", subcore_axis_name="s")


def _in_start(h_hbm, cid, buf, sem):
    pltpu.async_copy(h_hbm.at[pl.ds(cid * CH, CH)], buf, sem)


def _in_wait(h_hbm, buf, sem):
    pltpu.make_async_copy(h_hbm.at[pl.ds(0, CH)], buf, sem).wait()


def _p1sc_body(h_hbm, w_hbm, out_hbm, wbuf, buf0, buf1, stage, sem0, sem1):
    wid = lax.axis_index("s") * NC + lax.axis_index("c")
    pltpu.sync_copy(w_hbm, wbuf)
    wv = [wbuf[pl.ds(t * L, L)] for t in range(T)]
    start = _worker_start(wid)

    def rows(buf, acc):
        def row_body(r, acc):
            acc = list(acc)
            for p in range(P):
                a = acc[p]
                for t in range(T):
                    a = a + buf[r, p, pl.ds(t * L, L)] * wv[t]
                acc[p] = a
            return tuple(acc)
        return lax.fori_loop(0, CH, row_body, acc)

    _in_start(h_hbm, start, buf0, sem0)

    def body2(k, acc):
        g = 2 * k
        _in_start(h_hbm, start + g + 1, buf1, sem1)
        _in_wait(h_hbm, buf0, sem0)
        acc = rows(buf0, acc)
        _in_start(h_hbm, start + g + 2, buf0, sem0)
        _in_wait(h_hbm, buf1, sem1)
        return rows(buf1, acc)

    acc0 = tuple(jnp.zeros((L,), jnp.float32) for _ in range(P))
    acc = lax.fori_loop(0, HALF, body2, acc0)
    _in_wait(h_hbm, buf0, sem0)
    acc = rows(buf0, acc)
    for p in range(P):
        stage[p, :] = acc[p]

    @pl.when(wid < SC_EXTRA)
    def _extra():
        pltpu.sync_copy(h_hbm.at[pl.ds((start + SC_BASE) * CH, CH)], buf1)
        acc_e = rows(buf1, acc0)
        for p in range(P):
            stage[p, :] = stage[p, :] + acc_e[p]

    pltpu.sync_copy(stage, out_hbm.at[wid])


def _p1sc(h, Wf):
    f = pl.kernel(
        _p1sc_body,
        out_type=jax.ShapeDtypeStruct((NW, P, L), jnp.float32),
        mesh=_sc_mesh(),
        scratch_types=[
            pltpu.VMEM((D,), jnp.float32),
            pltpu.VMEM((CH, P, D), jnp.float32),
            pltpu.VMEM((CH, P, D), jnp.float32),
            pltpu.VMEM((P, L), jnp.float32),
            pltpu.SemaphoreType.DMA,
            pltpu.SemaphoreType.DMA,
        ],
    )
    return f(h, Wf)


def _p1tc_body(h_ref, zphi_ref, acc_ref):
    i = pl.program_id(0)

    @pl.when(i == 0)
    def _init():
        acc_ref[...] = jnp.zeros_like(acc_ref)

    acc_ref[...] += jnp.sum(h_ref[...], axis=0)

    @pl.when(i == NB - 1)
    def _fin():
        zphi_ref[...] = acc_ref[...]


def _p1tc(h):
    return pl.pallas_call(
        _p1tc_body,
        grid=(NB,),
        in_specs=[pl.BlockSpec((BN, P, D), lambda i: (i, 0, 0))],
        out_specs=pl.BlockSpec((P, D), lambda i: (0, 0)),
        out_shape=jax.ShapeDtypeStruct((P, D), jnp.float32),
        scratch_shapes=[pltpu.VMEM((P, D), jnp.float32)],
    )(h)


def _p2tc_body(zphi_ref, w_ref, wpart_ref, h_ref, z_ref, beta_ref):
    i = pl.program_id(0)

    @pl.when(i == 0)
    def _betastep():
        w = jnp.dot(zphi_ref[...], w_ref[...])
        w = w + jnp.sum(wpart_ref[...], axis=(0, 2)).reshape(P, 1)
        w = jnp.where(w >= 0, w, 0.01 * w)
        m = jnp.max(w, axis=0, keepdims=True)
        e = jnp.exp(w - m)
        beta = e / jnp.sum(e, axis=0, keepdims=True)
        beta_ref[...] = jnp.broadcast_to(beta, (P, D))

    z_ref[...] = jnp.sum(h_ref[...] * beta_ref[...][None, :, :], axis=1)


def _p2tc(zphi, Wm, wpart, h):
    return pl.pallas_call(
        _p2tc_body,
        grid=(NB2,),
        in_specs=[
            pl.BlockSpec((P, D), lambda i: (0, 0)),
            pl.BlockSpec((D, 1), lambda i: (0, 0)),
            pl.BlockSpec((NW, P, L), lambda i: (0, 0, 0)),
            pl.BlockSpec((BN2, P, D), lambda i: (i, 0, 0)),
        ],
        out_specs=pl.BlockSpec((BN2, D), lambda i: (i, 0)),
        out_shape=jax.ShapeDtypeStruct((N, D), jnp.float32),
        scratch_shapes=[pltpu.VMEM((P, D), jnp.float32)],
    )(zphi, Wm, wpart, h)


def kernel(h, W):
    wpart = _p1sc(h, W.reshape(D))
    zphi = _p1tc(h)
    return _p2tc(zphi, W, wpart, h)
